# baseline (speedup 1.0000x reference)
import functools

import jax
import jax.numpy as jnp
from jax.experimental import pallas as pl
from jax.experimental.pallas import tpu as pltpu


def _expert_mlp_kernel(x_ref, wi_ref, wo_ref, m0_ref, m1_ref, r0_ref, r1_ref,
                       o_ref, xb_ref, *, bt, nt):
    e = pl.program_id(0)
    f = pl.program_id(1)
    first = (e == 0) & (f == 0)

    @pl.when(first)
    def _():
        xb_ref[...] = x_ref[...].astype(jnp.bfloat16)

    wib = wi_ref[0].astype(jnp.bfloat16)
    wob = wo_ref[0].astype(jnp.bfloat16)

    for t in range(nt):
        rows = pl.ds(t * bt, bt)
        x = xb_ref[rows, :]
        h = jax.lax.dot_general(x, wib, (((1,), (1,)), ((), ())),
                                preferred_element_type=jnp.float32)
        h = jnp.maximum(h, 0.0).astype(jnp.bfloat16)
        o = jax.lax.dot_general(h, wob, (((1,), (1,)), ((), ())),
                                preferred_element_type=jnp.float32)

        call = (m0_ref[rows, :] * r0_ref[rows, :]
                + m1_ref[rows, :] * r1_ref[rows, :])
        onehot = jax.lax.broadcasted_iota(jnp.int32, call.shape, 1) == e
        c = jnp.sum(jnp.where(onehot, call, 0.0), axis=1, keepdims=True)
        contrib = o * c

        @pl.when(first)
        def _():
            o_ref[rows, :] = contrib

        @pl.when(jnp.logical_not(first))
        def _():
            o_ref[rows, :] += contrib


def kernel(hidden_states, selected_experts, routing_weights, wi, wo):
    T, D = hidden_states.shape
    E, F, _ = wi.shape

    maskf = selected_experts.astype(jnp.float32)
    m0 = maskf[:, 0, :]
    m1 = maskf[:, 1, :]
    r0 = routing_weights[:, 0:1]
    r1 = routing_weights[:, 1:2]

    BT = 1024
    BF = 768
    NT = T // BT
    NF = F // BF

    body = functools.partial(_expert_mlp_kernel, bt=BT, nt=NT)

    out = pl.pallas_call(
        body,
        grid=(E, NF),
        in_specs=[
            pl.BlockSpec((T, D), lambda e, f: (0, 0)),
            pl.BlockSpec((1, BF, D), lambda e, f: (e, f, 0)),
            pl.BlockSpec((1, D, BF), lambda e, f: (e, 0, f)),
            pl.BlockSpec((T, E), lambda e, f: (0, 0)),
            pl.BlockSpec((T, E), lambda e, f: (0, 0)),
            pl.BlockSpec((T, 1), lambda e, f: (0, 0)),
            pl.BlockSpec((T, 1), lambda e, f: (0, 0)),
        ],
        out_specs=pl.BlockSpec((T, D), lambda e, f: (0, 0)),
        out_shape=jax.ShapeDtypeStruct((T, D), jnp.float32),
        scratch_shapes=[pltpu.VMEM((T, D), jnp.bfloat16)],
    )(hidden_states, wi, wo, m0, m1, r0, r1)
    return out
